# baseline (speedup 1.0000x reference)
import functools

import jax
import jax.numpy as jnp
from jax.experimental import pallas as pl

NUM_CLASSES = 8
INPUT_DIM = 34
EMB = 128
WIDTH = 126
SPACE = 4
PROP = 32
K = 32
B = 8
S = 2048
N = B * S
BLK = 256
NBLK = N // BLK
QBLK = S // BLK

NEG = -1e30


def _dotd(a, b):
    return jnp.dot(a.astype(jnp.bfloat16), b.astype(jnp.bfloat16),
                   preferred_element_type=jnp.float32)

def _elu(t):
    return jnp.where(t > 0, t, jnp.exp(jnp.minimum(t, 0.0)) - 1.0)


def _ln(t, g, b):
    mu = jnp.mean(t, axis=-1, keepdims=True)
    var = jnp.mean((t - mu) ** 2, axis=-1, keepdims=True)
    return (t - mu) / jnp.sqrt(var + 1e-5) * g + b


def _full_spec(shape):
    nd = len(shape)
    return pl.BlockSpec(shape, lambda *_: (0,) * nd)


def _row_spec(d):
    return pl.BlockSpec((BLK, d), lambda i: (i, 0))


def _encoder_body(x_ref, w0, b0, w1, b1, w2, b2, w3, b3, o_ref):
    t = x_ref[...]
    t = _elu(_dotd(t, w0[...]) + b0[...])
    t = _elu(_dotd(t, w1[...]) + b1[...])
    t = _elu(_dotd(t, w2[...]) + b2[...])
    o_ref[...] = _dotd(t, w3[...]) + b3[...]


def _encoder(x, nn0):
    ws = []
    specs = [_row_spec(INPUT_DIM)]
    for lin in nn0:
        w = lin['W']
        b = lin['b'].reshape(1, -1)
        ws += [w, b]
        specs += [_full_spec(w.shape), _full_spec(b.shape)]
    return pl.pallas_call(
        _encoder_body,
        grid=(NBLK,),
        in_specs=specs,
        out_specs=_row_spec(EMB),
        out_shape=jax.ShapeDtypeStruct((N, EMB), jnp.float32),
    )(x, *ws)


def _proj_body(x_ref, ws, bs, wh, bh, s_ref, h_ref):
    t = x_ref[...]
    s_ref[...] = _dotd(t, ws[...]) + bs[...]
    h_ref[...] = _dotd(t, wh[...]) + bh[...]


def _proj(x, p):
    ws, bs = p['lin_s']['W'], p['lin_s']['b'].reshape(1, -1)
    wh, bh = p['lin_h']['W'], p['lin_h']['b'].reshape(1, -1)
    return pl.pallas_call(
        _proj_body,
        grid=(NBLK,),
        in_specs=[_row_spec(EMB), _full_spec(ws.shape), _full_spec(bs.shape),
                  _full_spec(wh.shape), _full_spec(bh.shape)],
        out_specs=[_row_spec(SPACE), _row_spec(PROP)],
        out_shape=[jax.ShapeDtypeStruct((N, SPACE), jnp.float32),
                   jax.ShapeDtypeStruct((N, PROP), jnp.float32)],
    )(x, ws, bs, wh, bh)


def _agg_body(sq_ref, sT_ref, hT_ref, o_ref):
    sq = sq_ref[0]
    sT = sT_ref[0]
    hT = hT_ref[0]

    snq = jnp.sum(sq * sq, axis=1, keepdims=True)
    sn = jnp.sum(sT * sT, axis=0, keepdims=True)
    cross = jax.lax.dot_general(sq.astype(jnp.bfloat16), sT.astype(jnp.bfloat16),
                                (((1,), (0,)), ((), ())),
                                preferred_element_type=jnp.float32)
    d2 = snq + sn - 2.0 * cross

    sn_max = jnp.max(sn, axis=1, keepdims=True)
    scale = snq + sn_max
    lo = -(0.05 * scale + 1e-3)
    hi = 2.1 * scale + 1e-3
    def it(_, c):
        lo_, hi_ = c
        mid = 0.5 * (lo_ + hi_)
        cnt = jnp.sum((d2 <= mid).astype(jnp.float32), axis=1, keepdims=True)
        ge = cnt >= K
        return jnp.where(ge, lo_, mid), jnp.where(ge, mid, hi_)

    lo, hi = jax.lax.fori_loop(0, 20, it, (lo, hi))
    mask = d2 <= hi
    w = jnp.exp(-10.0 * d2)
    a = jnp.where(mask, w, 0.0)
    maskbig = jnp.where(mask, 0.0, NEG)

    mean = jax.lax.dot_general(a, hT, (((1,), (1,)), ((), ())),
                               preferred_element_type=jnp.float32,
                               precision=jax.lax.Precision.HIGHEST) * (1.0 / K)

    cols = []
    for p in range(PROP):
        hrow = hT[p:p + 1, :]
        m1 = jnp.max(a * hrow, axis=1, keepdims=True)
        col = jax.lax.cond(
            jnp.min(m1) > 0.0,
            lambda m1=m1: m1,
            lambda hrow=hrow: jnp.max(a * hrow + maskbig, axis=1, keepdims=True),
        )
        cols.append(col)
    mx = jnp.concatenate(cols, axis=1)
    o_ref[...] = jnp.concatenate([mean, mx], axis=1)


def _agg(s3, sT3, hT3):
    return pl.pallas_call(
        _agg_body,
        grid=(B, QBLK),
        in_specs=[
            pl.BlockSpec((1, BLK, SPACE), lambda b, q: (b, q, 0)),
            pl.BlockSpec((1, SPACE, S), lambda b, q: (b, 0, 0)),
            pl.BlockSpec((1, PROP, S), lambda b, q: (b, 0, 0)),
        ],
        out_specs=pl.BlockSpec((BLK, 2 * PROP), lambda b, q: (b * QBLK + q, 0)),
        out_shape=jax.ShapeDtypeStruct((N, 2 * PROP), jnp.float32),
    )(s3, sT3, hT3)


def _combine_body(x_ref, a_ref, w1, w2, b2, g, bb, o_ref):
    x = x_ref[...]
    t = _dotd(x, w1[...]) + _dotd(a_ref[...], w2[...]) + b2[...]
    o_ref[...] = _ln(x + t, g[...], bb[...])


def _combine(x, agg, p):
    w1 = p['lin_out1']['W']
    w2, b2 = p['lin_out2']['W'], p['lin_out2']['b'].reshape(1, -1)
    g, bb = p['norm1']['g'].reshape(1, -1), p['norm1']['b'].reshape(1, -1)
    return pl.pallas_call(
        _combine_body,
        grid=(NBLK,),
        in_specs=[_row_spec(EMB), _row_spec(2 * PROP), _full_spec(w1.shape),
                  _full_spec(w2.shape), _full_spec(b2.shape),
                  _full_spec(g.shape), _full_spec(bb.shape)],
        out_specs=_row_spec(EMB),
        out_shape=jax.ShapeDtypeStruct((N, EMB), jnp.float32),
    )(x, agg, w1, w2, b2, g, bb)


def _gravnet(x, p):
    s, h = _proj(x, p)
    s3 = s.reshape(B, S, SPACE)
    sT3 = jnp.swapaxes(s3, 1, 2)
    hT3 = jnp.swapaxes(h.reshape(B, S, PROP), 1, 2)
    agg = _agg(s3, sT3, hT3)
    return _combine(x, agg, p)


def _ffn_run(t_first, w0_chunks, b0, wm, bm, g, beta, wo, bo):
    t = _ln(_elu(t_first + b0), g[0], beta[0])
    for i in range(4):
        t = _dotd(t, wm[i]) + bm[i]
        t = _ln(_elu(t), g[i + 1], beta[i + 1])
    return _dotd(t, wo) + bo


def _decoder_body(x_ref, i0_ref, i1_ref, r0_ref, r1_ref,
                  iw0, ib0, iwm, ibm, ig, ibe, iwo, ibo,
                  rw0, rb0, rwm, rbm, rg, rbe, rwo, rbo,
                  id_ref, mom_ref, chg_ref):
    x = x_ref[...]
    i0 = i0_ref[...]
    i1 = i1_ref[...]
    r0 = r0_ref[...]
    r1 = r1_ref[...]

    dot = _dotd

    t0 = (dot(x, iw0[0:INPUT_DIM]) + dot(i0, iw0[INPUT_DIM:INPUT_DIM + EMB])
          + dot(i1, iw0[INPUT_DIM + EMB:INPUT_DIM + 2 * EMB]))
    pid = _ffn_run(t0, None, ib0[...], iwm[...], ibm[...], ig[...], ibe[...],
                   iwo[...], ibo[...])
    id_ref[...] = pid

    heads = []
    for j in range(5):
        w0 = rw0[j]
        t0 = (dot(x, w0[0:INPUT_DIM]) + dot(r0, w0[INPUT_DIM:INPUT_DIM + EMB])
              + dot(r1, w0[INPUT_DIM + EMB:INPUT_DIM + 2 * EMB])
              + dot(pid, w0[INPUT_DIM + 2 * EMB:]))
        heads.append(_ffn_run(t0, None, rb0[j], rwm[j], rbm[j], rg[j], rbe[j],
                              rwo[j], rbo[j]))

    pt = heads[0][:, 0:1] + x[:, 1:2]
    eta = heads[1][:, 0:1] + x[:, 2:3]
    phi = heads[2][:, 0:2] + x[:, 3:5]
    en = heads[3][:, 0:1] + x[:, 5:6]
    mom_ref[...] = jnp.concatenate([pt, eta, phi, en], axis=1)
    chg_ref[...] = heads[4][:, 0:3]


def _pack_ffn(p, out_pad):
    w0 = p['lins'][0]['W']
    b0 = p['lins'][0]['b'].reshape(1, -1)
    wm = jnp.stack([p['lins'][i]['W'] for i in range(1, 5)])
    bm = jnp.stack([p['lins'][i]['b'].reshape(1, -1) for i in range(1, 5)])
    g = jnp.stack([p['lns'][i]['g'].reshape(1, -1) for i in range(5)])
    be = jnp.stack([p['lns'][i]['b'].reshape(1, -1) for i in range(5)])
    wo = p['lins'][5]['W']
    bo = p['lins'][5]['b'].reshape(1, -1)
    dout = wo.shape[1]
    if dout < out_pad:
        wo = jnp.pad(wo, ((0, 0), (0, out_pad - dout)))
        bo = jnp.pad(bo, ((0, 0), (0, out_pad - dout)))
    return [w0, b0, wm, bm, g, be, wo, bo]


def _decoder(x, i0, i1, r0, r1, params):
    id_parts = _pack_ffn(params['nn_id'], NUM_CLASSES)
    reg_parts = None
    for name in ('nn_pt', 'nn_eta', 'nn_phi', 'nn_energy', 'nn_charge'):
        parts = _pack_ffn(params[name], 8)
        if reg_parts is None:
            reg_parts = [[a] for a in parts]
        else:
            for acc, a in zip(reg_parts, parts):
                acc.append(a)
    reg_parts = [jnp.stack(a) for a in reg_parts]

    args = [x, i0, i1, r0, r1] + id_parts + reg_parts
    specs = ([_row_spec(INPUT_DIM)] + [_row_spec(EMB)] * 4
             + [_full_spec(a.shape) for a in id_parts + reg_parts])
    return pl.pallas_call(
        _decoder_body,
        grid=(NBLK,),
        in_specs=specs,
        out_specs=[_row_spec(NUM_CLASSES), _row_spec(5), _row_spec(3)],
        out_shape=[jax.ShapeDtypeStruct((N, NUM_CLASSES), jnp.float32),
                   jax.ShapeDtypeStruct((N, 5), jnp.float32),
                   jax.ShapeDtypeStruct((N, 3), jnp.float32)],
    )(*args)


def kernel(x, batch, params):
    xf = x.astype(jnp.float32)
    e = _encoder(xf, params['nn0'])

    def chain(plist):
        outs = []
        cur = e
        for p in plist:
            cur = _gravnet(cur, p)
            outs.append(cur)
        return outs

    i0, i1 = chain(params['conv_id'])
    r0, r1 = chain(params['conv_reg'])
    return _decoder(xf, i0, i1, r0, r1, params)


# speedup vs baseline: 2.0598x; 2.0598x over previous
import functools

import jax
import jax.numpy as jnp
from jax import lax
from jax.experimental import pallas as pl
from jax.experimental.pallas import tpu as pltpu
from jax.experimental.pallas import tpu_sc as plsc

NUM_CLASSES = 8
INPUT_DIM = 34
EMB = 128
WIDTH = 126
SPACE = 4
PROP = 32
K = 32
B = 8
S = 2048
N = B * S
BLK = 256
NBLK = N // BLK
QBLK = S // BLK

NEG = -1e30
NGRP = S // 16


def _dotd(a, b):
    return jnp.dot(a.astype(jnp.bfloat16), b.astype(jnp.bfloat16),
                   preferred_element_type=jnp.float32)

def _elu(t):
    return jnp.where(t > 0, t, jnp.exp(jnp.minimum(t, 0.0)) - 1.0)


def _ln(t, g, b):
    mu = jnp.mean(t, axis=-1, keepdims=True)
    var = jnp.mean((t - mu) ** 2, axis=-1, keepdims=True)
    return (t - mu) / jnp.sqrt(var + 1e-5) * g + b


def _full_spec(shape):
    nd = len(shape)
    return pl.BlockSpec(shape, lambda *_: (0,) * nd)


def _row_spec(d):
    return pl.BlockSpec((BLK, d), lambda i: (i, 0))


def _encoder_body(x_ref, w0, b0, w1, b1, w2, b2, w3, b3, o_ref):
    t = x_ref[...]
    t = _elu(_dotd(t, w0[...]) + b0[...])
    t = _elu(_dotd(t, w1[...]) + b1[...])
    t = _elu(_dotd(t, w2[...]) + b2[...])
    o_ref[...] = _dotd(t, w3[...]) + b3[...]


def _encoder(x, nn0):
    ws = []
    specs = [_row_spec(INPUT_DIM)]
    for lin in nn0:
        w = lin['W']
        b = lin['b'].reshape(1, -1)
        ws += [w, b]
        specs += [_full_spec(w.shape), _full_spec(b.shape)]
    return pl.pallas_call(
        _encoder_body,
        grid=(NBLK,),
        in_specs=specs,
        out_specs=_row_spec(EMB),
        out_shape=jax.ShapeDtypeStruct((N, EMB), jnp.float32),
    )(x, *ws)


def _proj_body(x_ref, ws, bs, wh, bh, s_ref, h_ref, s8_ref):
    t = x_ref[...]
    s = _dotd(t, ws[...]) + bs[...]
    s_ref[...] = s
    h_ref[...] = _dotd(t, wh[...]) + bh[...]
    sn = jnp.sum(s * s, axis=1, keepdims=True)
    s8_ref[...] = jnp.concatenate([s, sn, jnp.zeros((BLK, 3), jnp.float32)], axis=1)


def _proj(x, p):
    ws, bs = p['lin_s']['W'], p['lin_s']['b'].reshape(1, -1)
    wh, bh = p['lin_h']['W'], p['lin_h']['b'].reshape(1, -1)
    return pl.pallas_call(
        _proj_body,
        grid=(NBLK,),
        in_specs=[_row_spec(EMB), _full_spec(ws.shape), _full_spec(bs.shape),
                  _full_spec(wh.shape), _full_spec(bh.shape)],
        out_specs=[_row_spec(SPACE), _row_spec(PROP), _row_spec(8)],
        out_shape=[jax.ShapeDtypeStruct((N, SPACE), jnp.float32),
                   jax.ShapeDtypeStruct((N, PROP), jnp.float32),
                   jax.ShapeDtypeStruct((N, 8), jnp.float32)],
    )(x, ws, bs, wh, bh)


def _select_body(sq_ref, sT_ref, o_ref):
    sq = sq_ref[0]
    sT = sT_ref[0]

    snq = jnp.sum(sq * sq, axis=1, keepdims=True)
    sn = jnp.sum(sT * sT, axis=0, keepdims=True)
    cross = jax.lax.dot_general(sq.astype(jnp.bfloat16), sT.astype(jnp.bfloat16),
                                (((1,), (0,)), ((), ())),
                                preferred_element_type=jnp.float32)
    d2 = snq + sn - 2.0 * cross

    sn_max = jnp.max(sn, axis=1, keepdims=True)
    scale = snq + sn_max
    lo = -(0.05 * scale + 1e-3)
    hi = 2.1 * scale + 1e-3
    def it(_, c):
        lo_, hi_ = c
        mid = 0.5 * (lo_ + hi_)
        cnt = jnp.sum((d2 <= mid).astype(jnp.float32), axis=1, keepdims=True)
        ge = cnt >= K
        return jnp.where(ge, lo_, mid), jnp.where(ge, mid, hi_)

    lo, hi = jax.lax.fori_loop(0, 20, it, (lo, hi))
    mask = d2 <= hi

    il = lax.broadcasted_iota(jnp.int32, (1, S), 1)
    tp = (jnp.int32(1) << (il % 16)).astype(jnp.float32)
    bits = jnp.where(mask, tp, 0.0).astype(jnp.bfloat16)
    r = lax.broadcasted_iota(jnp.int32, (S, NGRP), 0)
    c = lax.broadcasted_iota(jnp.int32, (S, NGRP), 1)
    gs = ((r // 16) == c).astype(jnp.bfloat16)
    words = jax.lax.dot_general(bits, gs, (((1,), (0,)), ((), ())),
                                preferred_element_type=jnp.float32)
    ones_sel = jnp.where(mask, 1.0, 0.0).astype(jnp.bfloat16)
    gx = ((r // 16) < c).astype(jnp.bfloat16)
    pfx = jax.lax.dot_general(ones_sel, gx, (((1,), (0,)), ((), ())),
                              preferred_element_type=jnp.float32)
    o_ref[...] = (words + pfx * 65536.0).astype(jnp.int32)


def _select(s3, sT3):
    return pl.pallas_call(
        _select_body,
        grid=(B, QBLK),
        in_specs=[
            pl.BlockSpec((1, BLK, SPACE), lambda b, q: (b, q, 0)),
            pl.BlockSpec((1, SPACE, S), lambda b, q: (b, 0, 0)),
        ],
        out_specs=pl.BlockSpec((BLK, NGRP), lambda b, q: (b * QBLK + q, 0)),
        out_shape=jax.ShapeDtypeStruct((N, NGRP), jnp.int32),
    )(s3, sT3)


NW = 32
QW = N // NW
QCB = 32


def _sc_agg(mw, s8, h):
    mesh = plsc.VectorSubcoreMesh(core_axis_name="c", subcore_axis_name="s")

    @functools.partial(
        pl.kernel, mesh=mesh,
        compiler_params=pltpu.CompilerParams(needs_layout_passes=False),
        out_type=jax.ShapeDtypeStruct((N * 2 * PROP,), jnp.float32),
        scratch_types=[
            pltpu.VMEM((S * PROP,), jnp.float32),
            pltpu.VMEM((S * 8,), jnp.float32),
            pltpu.VMEM((QCB * NGRP,), jnp.int32),
            pltpu.VMEM((QCB * 2 * PROP,), jnp.float32),
            pltpu.VMEM((80,), jnp.int32),
            pltpu.VMEM((80,), jnp.float32),
            pltpu.VMEM((80,), jnp.float32),
            pltpu.SemaphoreType.DMA,
        ])
    def body(mw_hbm, s8_hbm, h_hbm, out_hbm,
             h_v, s8_v, mw_v, out_v, idx_v, w_v, wn_v, sem):
        wid = lax.axis_index("s") * 2 + lax.axis_index("c")
        q0 = wid * QW
        ev_base = (q0 // S) * S
        pltpu.sync_copy(h_hbm.at[pl.ds(ev_base * PROP, S * PROP)], h_v)
        pltpu.sync_copy(s8_hbm.at[pl.ds(ev_base * 8, S * 8)], s8_v)
        zero16i = jnp.zeros((16,), jnp.int32)
        lane = lax.iota(jnp.int32, 16)
        negv = jnp.full((16,), NEG, jnp.float32)
        zerov = jnp.zeros((16,), jnp.float32)
        for t in range(5):
            idx_v[pl.ds(t * 16, 16)] = zero16i

        def q_body(qi, cbase):
            qloc = (cbase - ev_base) + qi

            cur = 0
            for g in range(NGRP // 16):
                wvv = mw_v[pl.ds(qi * NGRP + g * 16, 16)]
                bitsv = wvv & 0xFFFF
                pfxv = lax.shift_right_logical(wvv, 16)
                base = (lane + g * 16) * 16
                off = zero16i
                for bb in range(16):
                    m = (bitsv & (1 << bb)) != 0
                    dest = jnp.minimum(jnp.where(m, pfxv + off, 79), 79)
                    plsc.store_scatter(idx_v, [dest], base + bb)
                    off = off + m.astype(jnp.int32)
                if g == NGRP // 16 - 1:
                    cur = pfxv[15] + off[15]
            cur = jnp.minimum(cur, 48)


            def bf(x):
                u = plsc.bitcast(x, jnp.int32)
                r = (u + 0x7FFF + (lax.shift_right_logical(u, 16) & 1)) & -65536
                return plsc.bitcast(r, jnp.float32)

            own = plsc.load_gather(s8_v, [zero16i + qloc * 8 + (lane & 7)])
            ownb = bf(own)
            b0, b1, b2, b3 = ownb[0], ownb[1], ownb[2], ownb[3]
            sin = own[4]
            for t in range(3):
                valid = (lane + t * 16) < cur
                iv = jnp.where(valid, idx_v[pl.ds(t * 16, 16)], 0)
                iv8 = iv * 8
                g0 = bf(plsc.load_gather(s8_v, [iv8]))
                g1 = bf(plsc.load_gather(s8_v, [iv8 + 1]))
                g2 = bf(plsc.load_gather(s8_v, [iv8 + 2]))
                g3 = bf(plsc.load_gather(s8_v, [iv8 + 3]))
                gnf = plsc.load_gather(s8_v, [iv8 + 4])
                acc = b0 * g0 + b1 * g1
                acc = acc + b2 * g2
                acc = acc + b3 * g3
                d2v = (sin + gnf) - 2.0 * acc
                wv = jnp.exp(-10.0 * d2v)
                w_v[pl.ds(t * 16, 16)] = jnp.where(valid, wv, 0.0)
                wn_v[pl.ds(t * 16, 16)] = jnp.where(valid, 0.0, NEG)

            me0 = zerov
            me1 = zerov
            mx0 = negv
            mx1 = negv
            for t in range(3):
                iv16 = idx_v[pl.ds(t * 16, 16)]
                wv16 = w_v[pl.ds(t * 16, 16)]
                wn16 = wn_v[pl.ds(t * 16, 16)]
                for l in range(16):
                    ij = iv16[l]
                    wj = wv16[l]
                    pj = wn16[l]
                    h0 = h_v[pl.ds(ij * PROP, 16)]
                    h1 = h_v[pl.ds(ij * PROP + 16, 16)]
                    m0 = wj * h0
                    m1 = wj * h1
                    me0 = me0 + m0
                    me1 = me1 + m1
                    mx0 = jnp.maximum(mx0, m0 + pj)
                    mx1 = jnp.maximum(mx1, m1 + pj)
            ob = qi * (2 * PROP)
            out_v[pl.ds(ob, 16)] = me0 * (1.0 / K)
            out_v[pl.ds(ob + 16, 16)] = me1 * (1.0 / K)
            out_v[pl.ds(ob + 32, 16)] = mx0
            out_v[pl.ds(ob + 48, 16)] = mx1
            return cbase

        def chunk_body(ci, _):
            qbase = q0 + ci * QCB
            pltpu.sync_copy(mw_hbm.at[pl.ds(qbase * NGRP, QCB * NGRP)], mw_v)
            lax.fori_loop(0, QCB, q_body, qbase)
            pltpu.sync_copy(out_v, out_hbm.at[pl.ds(qbase * 2 * PROP, QCB * 2 * PROP)])
            return 0

        lax.fori_loop(0, QW // QCB, chunk_body, 0)

    return body(mw, s8, h)


def _combine_body(x_ref, a_ref, w1, w2, b2, g, bb, o_ref):
    x = x_ref[...]
    t = _dotd(x, w1[...]) + _dotd(a_ref[...], w2[...]) + b2[...]
    o_ref[...] = _ln(x + t, g[...], bb[...])


def _combine(x, agg, p):
    w1 = p['lin_out1']['W']
    w2, b2 = p['lin_out2']['W'], p['lin_out2']['b'].reshape(1, -1)
    g, bb = p['norm1']['g'].reshape(1, -1), p['norm1']['b'].reshape(1, -1)
    return pl.pallas_call(
        _combine_body,
        grid=(NBLK,),
        in_specs=[_row_spec(EMB), _row_spec(2 * PROP), _full_spec(w1.shape),
                  _full_spec(w2.shape), _full_spec(b2.shape),
                  _full_spec(g.shape), _full_spec(bb.shape)],
        out_specs=_row_spec(EMB),
        out_shape=jax.ShapeDtypeStruct((N, EMB), jnp.float32),
    )(x, agg, w1, w2, b2, g, bb)


def _gravnet(x, p):
    s, h, s8 = _proj(x, p)
    s3 = s.reshape(B, S, SPACE)
    sT3 = jnp.swapaxes(s3, 1, 2)
    mw = _select(s3, sT3)
    agg = _sc_agg(mw.reshape(-1), s8.reshape(-1), h.reshape(-1)).reshape(N, 2 * PROP)
    return _combine(x, agg, p)


def _ffn_run(t_first, w0_chunks, b0, wm, bm, g, beta, wo, bo):
    t = _ln(_elu(t_first + b0), g[0], beta[0])
    for i in range(4):
        t = _dotd(t, wm[i]) + bm[i]
        t = _ln(_elu(t), g[i + 1], beta[i + 1])
    return _dotd(t, wo) + bo


def _decoder_body(x_ref, i0_ref, i1_ref, r0_ref, r1_ref,
                  iw0, ib0, iwm, ibm, ig, ibe, iwo, ibo,
                  rw0, rb0, rwm, rbm, rg, rbe, rwo, rbo,
                  id_ref, mom_ref, chg_ref):
    x = x_ref[...]
    i0 = i0_ref[...]
    i1 = i1_ref[...]
    r0 = r0_ref[...]
    r1 = r1_ref[...]

    dot = _dotd

    t0 = (dot(x, iw0[0:INPUT_DIM]) + dot(i0, iw0[INPUT_DIM:INPUT_DIM + EMB])
          + dot(i1, iw0[INPUT_DIM + EMB:INPUT_DIM + 2 * EMB]))
    pid = _ffn_run(t0, None, ib0[...], iwm[...], ibm[...], ig[...], ibe[...],
                   iwo[...], ibo[...])
    id_ref[...] = pid

    heads = []
    for j in range(5):
        w0 = rw0[j]
        t0 = (dot(x, w0[0:INPUT_DIM]) + dot(r0, w0[INPUT_DIM:INPUT_DIM + EMB])
              + dot(r1, w0[INPUT_DIM + EMB:INPUT_DIM + 2 * EMB])
              + dot(pid, w0[INPUT_DIM + 2 * EMB:]))
        heads.append(_ffn_run(t0, None, rb0[j], rwm[j], rbm[j], rg[j], rbe[j],
                              rwo[j], rbo[j]))

    pt = heads[0][:, 0:1] + x[:, 1:2]
    eta = heads[1][:, 0:1] + x[:, 2:3]
    phi = heads[2][:, 0:2] + x[:, 3:5]
    en = heads[3][:, 0:1] + x[:, 5:6]
    mom_ref[...] = jnp.concatenate([pt, eta, phi, en], axis=1)
    chg_ref[...] = heads[4][:, 0:3]


def _pack_ffn(p, out_pad):
    w0 = p['lins'][0]['W']
    b0 = p['lins'][0]['b'].reshape(1, -1)
    wm = jnp.stack([p['lins'][i]['W'] for i in range(1, 5)])
    bm = jnp.stack([p['lins'][i]['b'].reshape(1, -1) for i in range(1, 5)])
    g = jnp.stack([p['lns'][i]['g'].reshape(1, -1) for i in range(5)])
    be = jnp.stack([p['lns'][i]['b'].reshape(1, -1) for i in range(5)])
    wo = p['lins'][5]['W']
    bo = p['lins'][5]['b'].reshape(1, -1)
    dout = wo.shape[1]
    if dout < out_pad:
        wo = jnp.pad(wo, ((0, 0), (0, out_pad - dout)))
        bo = jnp.pad(bo, ((0, 0), (0, out_pad - dout)))
    return [w0, b0, wm, bm, g, be, wo, bo]


def _decoder(x, i0, i1, r0, r1, params):
    id_parts = _pack_ffn(params['nn_id'], NUM_CLASSES)
    reg_parts = None
    for name in ('nn_pt', 'nn_eta', 'nn_phi', 'nn_energy', 'nn_charge'):
        parts = _pack_ffn(params[name], 8)
        if reg_parts is None:
            reg_parts = [[a] for a in parts]
        else:
            for acc, a in zip(reg_parts, parts):
                acc.append(a)
    reg_parts = [jnp.stack(a) for a in reg_parts]

    args = [x, i0, i1, r0, r1] + id_parts + reg_parts
    specs = ([_row_spec(INPUT_DIM)] + [_row_spec(EMB)] * 4
             + [_full_spec(a.shape) for a in id_parts + reg_parts])
    return pl.pallas_call(
        _decoder_body,
        grid=(NBLK,),
        in_specs=specs,
        out_specs=[_row_spec(NUM_CLASSES), _row_spec(5), _row_spec(3)],
        out_shape=[jax.ShapeDtypeStruct((N, NUM_CLASSES), jnp.float32),
                   jax.ShapeDtypeStruct((N, 5), jnp.float32),
                   jax.ShapeDtypeStruct((N, 3), jnp.float32)],
    )(*args)


def kernel(x, batch, params):
    xf = x.astype(jnp.float32)
    e = _encoder(xf, params['nn0'])

    def chain(plist):
        outs = []
        cur = e
        for p in plist:
            cur = _gravnet(cur, p)
            outs.append(cur)
        return outs

    i0, i1 = chain(params['conv_id'])
    r0, r1 = chain(params['conv_reg'])
    return _decoder(xf, i0, i1, r0, r1, params)


# speedup vs baseline: 2.0620x; 1.0010x over previous
import functools

import jax
import jax.numpy as jnp
from jax import lax
from jax.experimental import pallas as pl
from jax.experimental.pallas import tpu as pltpu
from jax.experimental.pallas import tpu_sc as plsc

NUM_CLASSES = 8
INPUT_DIM = 34
EMB = 128
WIDTH = 126
SPACE = 4
PROP = 32
K = 32
B = 8
S = 2048
N = B * S
BLK = 256
NBLK = N // BLK
QBLK = S // BLK

NEG = -1e30
NGRP = S // 16


def _dotd(a, b):
    return jnp.dot(a.astype(jnp.bfloat16), b.astype(jnp.bfloat16),
                   preferred_element_type=jnp.float32)

def _elu(t):
    return jnp.where(t > 0, t, jnp.exp(jnp.minimum(t, 0.0)) - 1.0)


def _ln(t, g, b):
    mu = jnp.mean(t, axis=-1, keepdims=True)
    var = jnp.mean((t - mu) ** 2, axis=-1, keepdims=True)
    return (t - mu) / jnp.sqrt(var + 1e-5) * g + b


def _full_spec(shape):
    nd = len(shape)
    return pl.BlockSpec(shape, lambda *_: (0,) * nd)


def _row_spec(d):
    return pl.BlockSpec((BLK, d), lambda i: (i, 0))


def _encoder_body(x_ref, w0, b0, w1, b1, w2, b2, w3, b3, o_ref):
    t = x_ref[...]
    t = _elu(_dotd(t, w0[...]) + b0[...])
    t = _elu(_dotd(t, w1[...]) + b1[...])
    t = _elu(_dotd(t, w2[...]) + b2[...])
    o_ref[...] = _dotd(t, w3[...]) + b3[...]


def _encoder(x, nn0):
    ws = []
    specs = [_row_spec(INPUT_DIM)]
    for lin in nn0:
        w = lin['W']
        b = lin['b'].reshape(1, -1)
        ws += [w, b]
        specs += [_full_spec(w.shape), _full_spec(b.shape)]
    return pl.pallas_call(
        _encoder_body,
        grid=(NBLK,),
        in_specs=specs,
        out_specs=_row_spec(EMB),
        out_shape=jax.ShapeDtypeStruct((N, EMB), jnp.float32),
    )(x, *ws)


def _proj_body(x_ref, ws, bs, wh, bh, s_ref, h_ref, s8_ref):
    t = x_ref[...]
    s = _dotd(t, ws[...]) + bs[...]
    s_ref[...] = s
    h_ref[...] = _dotd(t, wh[...]) + bh[...]
    sn = jnp.sum(s * s, axis=1, keepdims=True)
    s8_ref[...] = jnp.concatenate([s, sn, jnp.zeros((BLK, 3), jnp.float32)], axis=1)


def _proj(x, p):
    ws, bs = p['lin_s']['W'], p['lin_s']['b'].reshape(1, -1)
    wh, bh = p['lin_h']['W'], p['lin_h']['b'].reshape(1, -1)
    return pl.pallas_call(
        _proj_body,
        grid=(NBLK,),
        in_specs=[_row_spec(EMB), _full_spec(ws.shape), _full_spec(bs.shape),
                  _full_spec(wh.shape), _full_spec(bh.shape)],
        out_specs=[_row_spec(SPACE), _row_spec(PROP), _row_spec(8)],
        out_shape=[jax.ShapeDtypeStruct((N, SPACE), jnp.float32),
                   jax.ShapeDtypeStruct((N, PROP), jnp.float32),
                   jax.ShapeDtypeStruct((N, 8), jnp.float32)],
    )(x, ws, bs, wh, bh)


def _select_body(sq_ref, sT_ref, o_ref):
    sq = sq_ref[0]
    sT = sT_ref[0]

    snq = jnp.sum(sq * sq, axis=1, keepdims=True)
    sn = jnp.sum(sT * sT, axis=0, keepdims=True)
    cross = jax.lax.dot_general(sq.astype(jnp.bfloat16), sT.astype(jnp.bfloat16),
                                (((1,), (0,)), ((), ())),
                                preferred_element_type=jnp.float32)
    d2 = snq + sn - 2.0 * cross

    sn_max = jnp.max(sn, axis=1, keepdims=True)
    scale = snq + sn_max
    lo = -(0.05 * scale + 1e-3)
    hi = 2.1 * scale + 1e-3
    def it(_, c):
        lo_, hi_ = c
        mid = 0.5 * (lo_ + hi_)
        cnt = jnp.sum((d2 <= mid).astype(jnp.float32), axis=1, keepdims=True)
        ge = cnt >= K
        return jnp.where(ge, lo_, mid), jnp.where(ge, mid, hi_)

    lo, hi = jax.lax.fori_loop(0, 20, it, (lo, hi))
    mask = d2 <= hi

    il = lax.broadcasted_iota(jnp.int32, (1, S), 1)
    tp = (jnp.int32(1) << (il % 16)).astype(jnp.float32)
    bits = jnp.where(mask, tp, 0.0).astype(jnp.bfloat16)
    r = lax.broadcasted_iota(jnp.int32, (S, NGRP), 0)
    c = lax.broadcasted_iota(jnp.int32, (S, NGRP), 1)
    gs = ((r // 16) == c).astype(jnp.bfloat16)
    words = jax.lax.dot_general(bits, gs, (((1,), (0,)), ((), ())),
                                preferred_element_type=jnp.float32)
    ones_sel = jnp.where(mask, 1.0, 0.0).astype(jnp.bfloat16)
    gx = ((r // 16) < c).astype(jnp.bfloat16)
    pfx = jax.lax.dot_general(ones_sel, gx, (((1,), (0,)), ((), ())),
                              preferred_element_type=jnp.float32)
    o_ref[...] = (words + pfx * 65536.0).astype(jnp.int32)


def _select(s3, sT3):
    return pl.pallas_call(
        _select_body,
        grid=(B, QBLK),
        in_specs=[
            pl.BlockSpec((1, BLK, SPACE), lambda b, q: (b, q, 0)),
            pl.BlockSpec((1, SPACE, S), lambda b, q: (b, 0, 0)),
        ],
        out_specs=pl.BlockSpec((BLK, NGRP), lambda b, q: (b * QBLK + q, 0)),
        out_shape=jax.ShapeDtypeStruct((N, NGRP), jnp.int32),
    )(s3, sT3)


NW = 32
QW = N // NW
QCB = 32


def _sc_agg(mw, s8, h):
    mesh = plsc.VectorSubcoreMesh(core_axis_name="c", subcore_axis_name="s")

    @functools.partial(
        pl.kernel, mesh=mesh,
        compiler_params=pltpu.CompilerParams(needs_layout_passes=False),
        out_type=jax.ShapeDtypeStruct((N * 2 * PROP,), jnp.float32),
        scratch_types=[
            pltpu.VMEM((S * PROP,), jnp.float32),
            pltpu.VMEM((S * 8,), jnp.float32),
            pltpu.VMEM((QCB * NGRP,), jnp.int32),
            pltpu.VMEM((QCB * 2 * PROP,), jnp.float32),
            pltpu.VMEM((80,), jnp.int32),
            pltpu.VMEM((80,), jnp.float32),
            pltpu.VMEM((80,), jnp.float32),
            pltpu.SemaphoreType.DMA,
        ])
    def body(mw_hbm, s8_hbm, h_hbm, out_hbm,
             h_v, s8_v, mw_v, out_v, idx_v, w_v, wn_v, sem):
        wid = lax.axis_index("s") * 2 + lax.axis_index("c")
        q0 = wid * QW
        ev_base = (q0 // S) * S
        pltpu.sync_copy(h_hbm.at[pl.ds(ev_base * PROP, S * PROP)], h_v)
        pltpu.sync_copy(s8_hbm.at[pl.ds(ev_base * 8, S * 8)], s8_v)
        zero16i = jnp.zeros((16,), jnp.int32)
        lane = lax.iota(jnp.int32, 16)
        negv = jnp.full((16,), NEG, jnp.float32)
        zerov = jnp.zeros((16,), jnp.float32)
        for t in range(5):
            idx_v[pl.ds(t * 16, 16)] = zero16i

        def q_body(qi, cbase):
            qloc = (cbase - ev_base) + qi

            cur = 0
            for g in range(NGRP // 16):
                wvv = mw_v[pl.ds(qi * NGRP + g * 16, 16)]
                bitsv = wvv & 0xFFFF
                pfxv = lax.shift_right_logical(wvv, 16)
                base = (lane + g * 16) * 16
                off = zero16i
                for bb in range(16):
                    m = (bitsv & (1 << bb)) != 0
                    dest = jnp.minimum(jnp.where(m, pfxv + off, 79), 79)
                    plsc.store_scatter(idx_v, [dest], base + bb)
                    off = off + m.astype(jnp.int32)
                if g == NGRP // 16 - 1:
                    cur = pfxv[15] + off[15]
            cur = jnp.minimum(cur, 48)


            def bf(x):
                u = plsc.bitcast(x, jnp.int32)
                r = (u + 0x7FFF + (lax.shift_right_logical(u, 16) & 1)) & -65536
                return plsc.bitcast(r, jnp.float32)

            own = plsc.load_gather(s8_v, [zero16i + qloc * 8 + (lane & 7)])
            ownb = bf(own)
            b0, b1, b2, b3 = ownb[0], ownb[1], ownb[2], ownb[3]
            sin = own[4]
            for t in range(3):
                valid = (lane + t * 16) < cur
                iv = jnp.where(valid, idx_v[pl.ds(t * 16, 16)], 0)
                iv8 = iv * 8
                g0 = bf(plsc.load_gather(s8_v, [iv8]))
                g1 = bf(plsc.load_gather(s8_v, [iv8 + 1]))
                g2 = bf(plsc.load_gather(s8_v, [iv8 + 2]))
                g3 = bf(plsc.load_gather(s8_v, [iv8 + 3]))
                gnf = plsc.load_gather(s8_v, [iv8 + 4])
                acc = b0 * g0 + b1 * g1
                acc = acc + b2 * g2
                acc = acc + b3 * g3
                d2v = (sin + gnf) - 2.0 * acc
                wv = jnp.exp(-10.0 * d2v)
                w_v[pl.ds(t * 16, 16)] = jnp.where(valid, wv, 0.0)
                wn_v[pl.ds(t * 16, 16)] = jnp.where(valid, 0.0, NEG)

            me0 = zerov
            me1 = zerov
            mx0 = negv
            mx1 = negv
            for t in range(3):
                iv16 = idx_v[pl.ds(t * 16, 16)]
                wv16 = w_v[pl.ds(t * 16, 16)]
                wn16 = wn_v[pl.ds(t * 16, 16)]
                for l in range(16):
                    ij = iv16[l]
                    wj = wv16[l]
                    pj = wn16[l]
                    h0 = h_v[pl.ds(ij * PROP, 16)]
                    h1 = h_v[pl.ds(ij * PROP + 16, 16)]
                    m0 = wj * h0
                    m1 = wj * h1
                    me0 = me0 + m0
                    me1 = me1 + m1
                    mx0 = jnp.maximum(mx0, m0 + pj)
                    mx1 = jnp.maximum(mx1, m1 + pj)
            ob = qi * (2 * PROP)
            out_v[pl.ds(ob, 16)] = me0 * (1.0 / K)
            out_v[pl.ds(ob + 16, 16)] = me1 * (1.0 / K)
            out_v[pl.ds(ob + 32, 16)] = mx0
            out_v[pl.ds(ob + 48, 16)] = mx1
            return cbase

        def chunk_body(ci, _):
            qbase = q0 + ci * QCB
            pltpu.sync_copy(mw_hbm.at[pl.ds(qbase * NGRP, QCB * NGRP)], mw_v)
            lax.fori_loop(0, QCB, q_body, qbase)
            pltpu.sync_copy(out_v, out_hbm.at[pl.ds(qbase * 2 * PROP, QCB * 2 * PROP)])
            return 0

        lax.fori_loop(0, QW // QCB, chunk_body, 0)

    return body(mw, s8, h)


def _combine_body(x_ref, a_ref, w1, w2, b2, g, bb, o_ref):
    x = x_ref[...]
    t = _dotd(x, w1[...]) + _dotd(a_ref[...], w2[...]) + b2[...]
    o_ref[...] = _ln(x + t, g[...], bb[...])


def _combine(x, agg, p):
    w1 = p['lin_out1']['W']
    w2, b2 = p['lin_out2']['W'], p['lin_out2']['b'].reshape(1, -1)
    g, bb = p['norm1']['g'].reshape(1, -1), p['norm1']['b'].reshape(1, -1)
    return pl.pallas_call(
        _combine_body,
        grid=(NBLK,),
        in_specs=[_row_spec(EMB), _row_spec(2 * PROP), _full_spec(w1.shape),
                  _full_spec(w2.shape), _full_spec(b2.shape),
                  _full_spec(g.shape), _full_spec(bb.shape)],
        out_specs=_row_spec(EMB),
        out_shape=jax.ShapeDtypeStruct((N, EMB), jnp.float32),
    )(x, agg, w1, w2, b2, g, bb)


def _gravnet_pair(xa, pa, xb, pb):
    sa, ha, s8a = _proj(xa, pa)
    sb_, hb, s8b = _proj(xb, pb)
    mwa = _select(sa.reshape(B, S, SPACE), jnp.swapaxes(sa.reshape(B, S, SPACE), 1, 2))
    agga = _sc_agg(mwa.reshape(-1), s8a.reshape(-1), ha.reshape(-1))
    mwb = _select(sb_.reshape(B, S, SPACE), jnp.swapaxes(sb_.reshape(B, S, SPACE), 1, 2))
    aggb = _sc_agg(mwb.reshape(-1), s8b.reshape(-1), hb.reshape(-1))
    ya = _combine(xa, agga.reshape(N, 2 * PROP), pa)
    yb = _combine(xb, aggb.reshape(N, 2 * PROP), pb)
    return ya, yb


def _ffn_run(t_first, w0_chunks, b0, wm, bm, g, beta, wo, bo):
    t = _ln(_elu(t_first + b0), g[0], beta[0])
    for i in range(4):
        t = _dotd(t, wm[i]) + bm[i]
        t = _ln(_elu(t), g[i + 1], beta[i + 1])
    return _dotd(t, wo) + bo


def _decoder_body(x_ref, i0_ref, i1_ref, r0_ref, r1_ref,
                  iw0, ib0, iwm, ibm, ig, ibe, iwo, ibo,
                  rw0, rb0, rwm, rbm, rg, rbe, rwo, rbo,
                  id_ref, mom_ref, chg_ref):
    x = x_ref[...]
    i0 = i0_ref[...]
    i1 = i1_ref[...]
    r0 = r0_ref[...]
    r1 = r1_ref[...]

    dot = _dotd

    t0 = (dot(x, iw0[0:INPUT_DIM]) + dot(i0, iw0[INPUT_DIM:INPUT_DIM + EMB])
          + dot(i1, iw0[INPUT_DIM + EMB:INPUT_DIM + 2 * EMB]))
    pid = _ffn_run(t0, None, ib0[...], iwm[...], ibm[...], ig[...], ibe[...],
                   iwo[...], ibo[...])
    id_ref[...] = pid

    heads = []
    for j in range(5):
        w0 = rw0[j]
        t0 = (dot(x, w0[0:INPUT_DIM]) + dot(r0, w0[INPUT_DIM:INPUT_DIM + EMB])
              + dot(r1, w0[INPUT_DIM + EMB:INPUT_DIM + 2 * EMB])
              + dot(pid, w0[INPUT_DIM + 2 * EMB:]))
        heads.append(_ffn_run(t0, None, rb0[j], rwm[j], rbm[j], rg[j], rbe[j],
                              rwo[j], rbo[j]))

    pt = heads[0][:, 0:1] + x[:, 1:2]
    eta = heads[1][:, 0:1] + x[:, 2:3]
    phi = heads[2][:, 0:2] + x[:, 3:5]
    en = heads[3][:, 0:1] + x[:, 5:6]
    mom_ref[...] = jnp.concatenate([pt, eta, phi, en], axis=1)
    chg_ref[...] = heads[4][:, 0:3]


def _pack_ffn(p, out_pad):
    w0 = p['lins'][0]['W']
    b0 = p['lins'][0]['b'].reshape(1, -1)
    wm = jnp.stack([p['lins'][i]['W'] for i in range(1, 5)])
    bm = jnp.stack([p['lins'][i]['b'].reshape(1, -1) for i in range(1, 5)])
    g = jnp.stack([p['lns'][i]['g'].reshape(1, -1) for i in range(5)])
    be = jnp.stack([p['lns'][i]['b'].reshape(1, -1) for i in range(5)])
    wo = p['lins'][5]['W']
    bo = p['lins'][5]['b'].reshape(1, -1)
    dout = wo.shape[1]
    if dout < out_pad:
        wo = jnp.pad(wo, ((0, 0), (0, out_pad - dout)))
        bo = jnp.pad(bo, ((0, 0), (0, out_pad - dout)))
    return [w0, b0, wm, bm, g, be, wo, bo]


def _decoder(x, i0, i1, r0, r1, params):
    id_parts = _pack_ffn(params['nn_id'], NUM_CLASSES)
    reg_parts = None
    for name in ('nn_pt', 'nn_eta', 'nn_phi', 'nn_energy', 'nn_charge'):
        parts = _pack_ffn(params[name], 8)
        if reg_parts is None:
            reg_parts = [[a] for a in parts]
        else:
            for acc, a in zip(reg_parts, parts):
                acc.append(a)
    reg_parts = [jnp.stack(a) for a in reg_parts]

    args = [x, i0, i1, r0, r1] + id_parts + reg_parts
    specs = ([_row_spec(INPUT_DIM)] + [_row_spec(EMB)] * 4
             + [_full_spec(a.shape) for a in id_parts + reg_parts])
    return pl.pallas_call(
        _decoder_body,
        grid=(NBLK,),
        in_specs=specs,
        out_specs=[_row_spec(NUM_CLASSES), _row_spec(5), _row_spec(3)],
        out_shape=[jax.ShapeDtypeStruct((N, NUM_CLASSES), jnp.float32),
                   jax.ShapeDtypeStruct((N, 5), jnp.float32),
                   jax.ShapeDtypeStruct((N, 3), jnp.float32)],
    )(*args)


def kernel(x, batch, params):
    xf = x.astype(jnp.float32)
    e = _encoder(xf, params['nn0'])

    i0, r0 = _gravnet_pair(e, params['conv_id'][0], e, params['conv_reg'][0])
    i1, r1 = _gravnet_pair(i0, params['conv_id'][1], r0, params['conv_reg'][1])
    return _decoder(xf, i0, i1, r0, r1, params)
